# baseline (speedup 1.0000x reference)
import jax
import jax.numpy as jnp
from jax.experimental import pallas as pl
from jax.experimental.pallas import tpu as pltpu

NSPLIT = 2


def _moe_body(x_ref, Wg_ref, bg_ref, We_ref, be_ref, out_ref, probs_ref):
    j = pl.program_id(0)
    e = pl.program_id(1)

    del j

    @pl.when(e == 0)
    def _init():
        logits = (
            jnp.dot(x_ref[...], Wg_ref[...], preferred_element_type=jnp.float32)
            + bg_ref[...]
        )
        m = jnp.max(logits, axis=-1, keepdims=True)
        p = jnp.exp(logits - m)
        probs_ref[...] = p / jnp.sum(p, axis=-1, keepdims=True)

    h = jnp.dot(x_ref[...], We_ref[0], preferred_element_type=jnp.float32)
    h = jnp.maximum(h + be_ref[0, 0], 0.0)

    lane = jax.lax.broadcasted_iota(jnp.int32, probs_ref.shape, 1)
    p_col = jnp.sum(
        jnp.where(lane == e, probs_ref[...], 0.0), axis=1, keepdims=True
    )
    contrib = h * p_col

    @pl.when(e == 0)
    def _first():
        out_ref[...] = contrib

    @pl.when(e > 0)
    def _rest():
        out_ref[...] += contrib


def kernel(x, Wg, bg, We, be):
    T, H = x.shape
    E = We.shape[0]
    nblk = H // NSPLIT
    bg2 = bg.reshape(1, E)
    be3 = be.reshape(E, 1, H)
    return pl.pallas_call(
        _moe_body,
        grid=(NSPLIT, E),
        in_specs=[
            pl.BlockSpec((T, H), lambda j, e: (0, 0)),
            pl.BlockSpec((H, E), lambda j, e: (0, 0)),
            pl.BlockSpec((1, E), lambda j, e: (0, 0)),
            pl.BlockSpec((1, H, nblk), lambda j, e: (e, 0, j)),
            pl.BlockSpec((1, 1, nblk), lambda j, e: (e, 0, j)),
        ],
        out_specs=pl.BlockSpec((T, nblk), lambda j, e: (0, j)),
        out_shape=jax.ShapeDtypeStruct((T, H), jnp.float32),
        scratch_shapes=[
            pltpu.VMEM((T, E), jnp.float32),
        ],
        compiler_params=pltpu.CompilerParams(
            dimension_semantics=("parallel", "arbitrary"),
        ),
    )(x, Wg, bg2, We, be3)


# speedup vs baseline: 1.0695x; 1.0695x over previous
import jax
import jax.numpy as jnp
from jax.experimental import pallas as pl
from jax.experimental.pallas import tpu as pltpu


def _moe_body(x_ref, Wg_ref, bg_ref, Wa_ref, Wb_ref, be_ref, out_ref, probs_ref):
    e = pl.program_id(0)

    @pl.when(e == 0)
    def _init():
        logits = (
            jnp.dot(x_ref[...], Wg_ref[...], preferred_element_type=jnp.float32)
            + bg_ref[...]
        )
        m = jnp.max(logits, axis=-1, keepdims=True)
        p = jnp.exp(logits - m)
        probs_ref[...] = p / jnp.sum(p, axis=-1, keepdims=True)

    kh = Wa_ref.shape[2]
    h = jnp.dot(x_ref[:, :kh], Wa_ref[0, 0], preferred_element_type=jnp.float32)
    h += jnp.dot(x_ref[:, kh:], Wb_ref[0, 0], preferred_element_type=jnp.float32)
    h = jnp.maximum(h + be_ref[0, 0], 0.0)

    lane = jax.lax.broadcasted_iota(jnp.int32, probs_ref.shape, 1)
    p_col = jnp.sum(
        jnp.where(lane == e, probs_ref[...], 0.0), axis=1, keepdims=True
    )
    contrib = h * p_col

    @pl.when(e == 0)
    def _first():
        out_ref[...] = contrib

    @pl.when(e > 0)
    def _rest():
        out_ref[...] += contrib


def kernel(x, Wg, bg, We, be):
    T, H = x.shape
    E = We.shape[0]
    bg2 = bg.reshape(1, E)
    be3 = be.reshape(E, 1, H)
    We4 = We.reshape(E, 2, H // 2, H)
    return pl.pallas_call(
        _moe_body,
        grid=(E,),
        in_specs=[
            pl.BlockSpec((T, H), lambda e: (0, 0)),
            pl.BlockSpec((H, E), lambda e: (0, 0)),
            pl.BlockSpec((1, E), lambda e: (0, 0)),
            pl.BlockSpec((1, 1, H // 2, H), lambda e: (e, 0, 0, 0)),
            pl.BlockSpec((1, 1, H // 2, H), lambda e: (e, 1, 0, 0)),
            pl.BlockSpec((1, 1, H), lambda e: (e, 0, 0)),
        ],
        out_specs=pl.BlockSpec((T, H), lambda e: (0, 0)),
        out_shape=jax.ShapeDtypeStruct((T, H), jnp.float32),
        scratch_shapes=[
            pltpu.VMEM((T, E), jnp.float32),
        ],
        compiler_params=pltpu.CompilerParams(
            dimension_semantics=("arbitrary",),
        ),
    )(x, Wg, bg2, We4, We4, be3)
